# baseline (speedup 1.0000x reference)
import functools

import jax
import jax.numpy as jnp
from jax import lax
from jax.experimental import pallas as pl
from jax.experimental.pallas import tpu as pltpu
from jax.experimental.pallas import tpu_sc as plsc

_B = 4
_N_IN = 12288
_N_OUT = 49152
_K = 4
_C = 32
_NC = 2
_NS = 16
_NW = _NC * _NS
_RPW = _N_OUT // _NW
_CHUNK = 16
_NSTEP = _RPW // _CHUNK
_G = _CHUNK * _K
_BC = _B * _C
_TROWS = _N_IN // 2
_IROWS = _RPW * _K // 128
_NBUF = 3


def _sc_body(xt, ci2, wf, out, xsh, idx_all, w_all, jl_v, g_v, o_v,
             gsem, osem):
    sid = lax.axis_index("s")
    wid = sid * _NC + lax.axis_index("c")
    base0 = wid * _RPW

    srows = _TROWS // _NS
    soff = pl.multiple_of(sid * srows, srows)
    pltpu.sync_copy(xt.at[pl.ds(soff, srows)], xsh.at[pl.ds(soff, srows)])

    pltpu.sync_copy(ci2.at[pl.ds(wid * _IROWS, _IROWS)], idx_all)
    pltpu.sync_copy(wf.at[pl.ds(base0 * _K, _RPW * _K)], w_all)

    plsc.subcore_barrier()

    def prep_and_gather(s, bi):
        for q in range(_G // 16):
            iv = idx_all[s // 2, pl.ds((s % 2) * _G + q * 16, 16)]
            jl_v[bi, pl.ds(q * 16, 16)] = lax.shift_right_logical(iv, 1)
        pltpu.async_copy(xsh.at[jl_v.at[bi]], g_v.at[bi], gsem.at[bi])

    prep_and_gather(0, 0)
    prep_and_gather(1, 1)

    def compute_step(s, bi):
        woff = s * (_CHUNK * _K)
        for i in range(_CHUNK // 4):
            wvec = w_all[pl.ds(woff + i * 16, 16)]
            pv = idx_all[s // 2, pl.ds((s % 2) * _G + i * 16, 16)]
            pvec = lax.shift_left(lax.bitwise_and(pv, 1), 6)
            for j in range(4):
                r = i * 4 + j
                e4 = r * _K
                ws = [wvec[j * _K + k] for k in range(_K)]
                ps = [pvec[j * _K + k] for k in range(_K)]
                for b in range(_B):
                    ve0, vo0 = plsc.unpack(
                        plsc.bitcast(
                            g_v[bi, e4, pl.ds(ps[0] + b * 16, 16)],
                            jnp.bfloat16,
                        ),
                        format=plsc.PackFormat.INTERLEAVED,
                    )
                    acc0 = ws[0] * ve0
                    acc1 = ws[0] * vo0
                    for k in range(1, _K):
                        ve, vo = plsc.unpack(
                            plsc.bitcast(
                                g_v[bi, e4 + k, pl.ds(ps[k] + b * 16, 16)],
                                jnp.bfloat16,
                            ),
                            format=plsc.PackFormat.INTERLEAVED,
                        )
                        acc0 = acc0 + ws[k] * ve
                        acc1 = acc1 + ws[k] * vo
                    o_v[bi, b, r, pl.ds(0, 16)] = acc0
                    o_v[bi, b, r, pl.ds(16, 16)] = acc1

    def tbody(p, carry):
        for t in range(_NBUF):
            s = p * _NBUF + t
            bi = t
            rbase = base0 + s * _CHUNK
            rbase_a = pl.multiple_of(rbase, _CHUNK)
            pltpu.make_async_copy(
                xt.at[pl.ds(0, _G)], g_v.at[bi], gsem.at[bi]
            ).wait()
            @pl.when(s >= _NBUF)
            def _():
                for b in range(_B):
                    pltpu.make_async_copy(
                        o_v.at[bi, b], out.at[b, pl.ds(rbase_a, _CHUNK)],
                        osem.at[bi],
                    ).wait()

            @pl.when(s + 2 < _NSTEP)
            def _():
                prep_and_gather(s + 2, (t + 2) % _NBUF)

            compute_step(s, bi)

            for b in range(_B):
                pltpu.async_copy(
                    o_v.at[bi, b], out.at[b, pl.ds(rbase_a, _CHUNK)],
                    osem.at[bi],
                )
        return carry

    lax.fori_loop(0, _NSTEP // _NBUF, tbody, 0)

    for sl in range(_NSTEP - _NBUF, _NSTEP):
        bi = sl % _NBUF
        rb_a = pl.multiple_of(base0 + sl * _CHUNK, _CHUNK)
        for b in range(_B):
            pltpu.make_async_copy(
                o_v.at[bi, b], out.at[b, pl.ds(rb_a, _CHUNK)], osem.at[bi]
            ).wait()


_upsample = functools.partial(
    pl.kernel,
    out_type=jax.ShapeDtypeStruct((_B, _N_OUT, _C), jnp.float32),
    mesh=plsc.VectorSubcoreMesh(core_axis_name="c", subcore_axis_name="s"),
    compiler_params=pltpu.CompilerParams(needs_layout_passes=False),
    scratch_types=[
        pltpu.VMEM_SHARED((_TROWS, 128), jnp.int32),
        pltpu.VMEM((_IROWS, 128), jnp.int32),
        pltpu.VMEM((_RPW * _K,), jnp.float32),
        pltpu.VMEM((_NBUF, _G), jnp.int32),
        pltpu.VMEM((_NBUF, _G, 128), jnp.int32),
        pltpu.VMEM((_NBUF, _B, _CHUNK, _C), jnp.float32),
        pltpu.SemaphoreType.DMA((_NBUF,)),
        pltpu.SemaphoreType.DMA((_NBUF,)),
    ],
)(_sc_body)


_CPERM = [c // 2 if c % 2 == 0 else 16 + c // 2 for c in range(_C)]


def kernel(x, connection_indices, interpolation_weights):
    xp = x[:, :, jnp.array(_CPERM, dtype=jnp.int32)]
    xbf = jnp.transpose(xp, (1, 0, 2)).reshape(_N_IN, _BC).astype(jnp.bfloat16)
    xt = jax.lax.bitcast_convert_type(
        xbf.reshape(_N_IN, _BC // 2, 2), jnp.int32
    ).reshape(_TROWS, 128)
    ci2 = connection_indices.reshape(_N_OUT * _K // 128, 128)
    wf = interpolation_weights.reshape(-1)
    return _upsample(xt, ci2, wf)


# speedup vs baseline: 1.4870x; 1.4870x over previous
import functools

import jax
import jax.numpy as jnp
from jax import lax
from jax.experimental import pallas as pl
from jax.experimental.pallas import tpu as pltpu
from jax.experimental.pallas import tpu_sc as plsc

_B = 4
_N_IN = 12288
_N_OUT = 49152
_K = 4
_C = 32
_NC = 2
_NS = 16
_NW = _NC * _NS
_RPW = _N_OUT // _NW
_CHUNK = 32
_NSTEP = _RPW // _CHUNK
_G = _CHUNK * _K
_IROWS = _RPW * _K // _G
_BC = _B * _C
_NBUF = 3


def _sc_body(xt, ci2, wf, out, idx_all, w_all, g_v, o_v, gsem, osem):
    wid = lax.axis_index("s") * _NC + lax.axis_index("c")
    base0 = wid * _RPW

    pltpu.sync_copy(ci2.at[pl.ds(wid * _IROWS, _IROWS)], idx_all)
    pltpu.sync_copy(wf.at[pl.ds(base0 * _K, _RPW * _K)], w_all)

    def start_gather(s, bi):
        pltpu.async_copy(xt.at[idx_all.at[s]], g_v.at[bi], gsem.at[bi])

    start_gather(0, 0)
    start_gather(1, 1)

    def compute_step(s, bi):
        woff = s * (_CHUNK * _K)
        for i in range(_CHUNK // 4):
            wvec = w_all[pl.ds(woff + i * 16, 16)]
            for j in range(4):
                r = i * 4 + j
                b4 = r * _K
                ws = [wvec[j * _K + k] for k in range(_K)]
                for b in range(_B):
                    for h in range(2):
                        col = b * _C + h * 16
                        acc = ws[0] * g_v[bi, b4, pl.ds(col, 16)]
                        for k in range(1, _K):
                            acc = acc + ws[k] * g_v[bi, b4 + k, pl.ds(col, 16)]
                        o_v[bi, b, r, pl.ds(h * 16, 16)] = acc

    def tbody(p, carry):
        for t in range(_NBUF):
            s = p * _NBUF + t
            bi = t
            rbase = base0 + s * _CHUNK
            rbase_a = pl.multiple_of(rbase, _CHUNK)
            pltpu.make_async_copy(
                xt.at[pl.ds(0, _G)], g_v.at[bi], gsem.at[bi]
            ).wait()
            @pl.when(s >= _NBUF)
            def _():
                for b in range(_B):
                    pltpu.make_async_copy(
                        o_v.at[bi, b], out.at[b, pl.ds(rbase_a, _CHUNK)],
                        osem.at[bi],
                    ).wait()

            @pl.when(s + 2 < _NSTEP)
            def _():
                start_gather(s + 2, (t + 2) % _NBUF)

            compute_step(s, bi)

            for b in range(_B):
                pltpu.async_copy(
                    o_v.at[bi, b], out.at[b, pl.ds(rbase_a, _CHUNK)], osem.at[bi]
                )
        return carry

    lax.fori_loop(0, _NSTEP // _NBUF, tbody, 0)

    for sl in range(_NSTEP - _NBUF, _NSTEP):
        bi = sl % _NBUF
        rb_a = pl.multiple_of(base0 + sl * _CHUNK, _CHUNK)
        for b in range(_B):
            pltpu.make_async_copy(
                o_v.at[bi, b], out.at[b, pl.ds(rb_a, _CHUNK)], osem.at[bi]
            ).wait()


_upsample = functools.partial(
    pl.kernel,
    out_type=jax.ShapeDtypeStruct((_B, _N_OUT, _C), jnp.float32),
    mesh=plsc.VectorSubcoreMesh(core_axis_name="c", subcore_axis_name="s"),
    scratch_types=[
        pltpu.VMEM((_IROWS, _G), jnp.int32),
        pltpu.VMEM((_RPW * _K,), jnp.float32),
        pltpu.VMEM((_NBUF, _G, _BC), jnp.float32),
        pltpu.VMEM((_NBUF, _B, _CHUNK, _C), jnp.float32),
        pltpu.SemaphoreType.DMA((_NBUF,)),
        pltpu.SemaphoreType.DMA((_NBUF,)),
    ],
)(_sc_body)


def kernel(x, connection_indices, interpolation_weights):
    xt = jnp.transpose(x, (1, 0, 2)).reshape(_N_IN, _BC)
    ci2 = connection_indices.reshape(_N_OUT * _K // _G, _G)
    wf = interpolation_weights.reshape(-1)
    return _upsample(xt, ci2, wf)
